# Initial kernel scaffold; baseline (speedup 1.0000x reference)
import jax
import jax.numpy as jnp
from jax.experimental import pallas as pl


def kernel(head_feature, tail_feature, adj, tmp_edge, edge_emb, W, W_e, a_l, a_r, a_e):
    raise NotImplementedError("write your pallas kernel here")


# speedup vs baseline: 28.3236x; 28.3236x over previous
import functools

import jax
import jax.numpy as jnp
from jax import lax
from jax.experimental import pallas as pl
from jax.experimental.pallas import tpu as pltpu
from jax.experimental.pallas import tpu_sc as plsc

N = 10000
E = 320000
D = 128
NC, NS = 2, 16
NW = NC * NS
EPW = E // NW
CH = 80
NCH = EPW // CH
SB = 640
SBL = N - (NS - 1) * SB

_f32 = jnp.float32
_i32 = jnp.int32

_MESH = plsc.VectorSubcoreMesh(core_axis_name="c", subcore_axis_name="s",
                               num_cores=NC, num_subcores=NS)


def _dense_body(hf_ref, tf_ref, wt_ref, al_ref, ar_ref, embt_ref, we_ref,
                ae_ref,
                ht_ref, hl_ref, hr_ref, he_ref, mub_ref, smax_ref):
    i = pl.program_id(0)
    ng = pl.num_programs(0)
    wt = wt_ref[...]
    ht = jnp.dot(tf_ref[...], wt, preferred_element_type=_f32)
    ht_ref[...] = ht
    hr = jnp.dot(ht, ar_ref[...], preferred_element_type=_f32)
    hr_ref[...] = hr
    wl = jnp.dot(wt, al_ref[...], preferred_element_type=_f32)
    hl = jnp.dot(hf_ref[...], wl, preferred_element_type=_f32)
    hl_ref[...] = hl
    cur = jnp.max(hl) + jnp.max(hr)

    @pl.when(i == 0)
    def _():
        smax_ref[0] = cur

    @pl.when(i > 0)
    def _():
        smax_ref[0] = jnp.maximum(smax_ref[0], cur)

    @pl.when(i == ng - 1)
    def _():
        e3 = jnp.dot(embt_ref[...], we_ref[...], preferred_element_type=_f32)
        he = jnp.sum(e3 * ae_ref[...], axis=1)
        he_ref[...] = he
        m = smax_ref[0] + jnp.max(he)
        mub_ref[0] = jnp.where(m > 0, m, 0.2 * m)


def _dense(hf, tf, wt, al, ar, embt, we, ae2):
    blk = 1000
    return pl.pallas_call(
        _dense_body,
        grid=(N // blk,),
        in_specs=[
            pl.BlockSpec((blk, D), lambda i: (i, 0)),
            pl.BlockSpec((blk, D), lambda i: (i, 0)),
            pl.BlockSpec((D, D), lambda i: (0, 0)),
            pl.BlockSpec((D, 1), lambda i: (0, 0)),
            pl.BlockSpec((D, 1), lambda i: (0, 0)),
            pl.BlockSpec((16, 16), lambda i: (0, 0)),
            pl.BlockSpec((16, 16), lambda i: (0, 0)),
            pl.BlockSpec((1, 16), lambda i: (0, 0)),
        ],
        out_specs=[
            pl.BlockSpec((blk, D), lambda i: (i, 0)),
            pl.BlockSpec((blk, 1), lambda i: (i, 0)),
            pl.BlockSpec((blk, 1), lambda i: (i, 0)),
            pl.BlockSpec((16,), lambda i: (0,)),
            pl.BlockSpec(memory_space=pltpu.SMEM),
            pl.BlockSpec(memory_space=pltpu.SMEM),
        ],
        out_shape=[
            jax.ShapeDtypeStruct((N, D), _f32),
            jax.ShapeDtypeStruct((N, 1), _f32),
            jax.ShapeDtypeStruct((N, 1), _f32),
            jax.ShapeDtypeStruct((16,), _f32),
            jax.ShapeDtypeStruct((1,), _f32),
            jax.ShapeDtypeStruct((1,), _f32),
        ],
    )(hf, tf, wt, al, ar, embt, we, ae2)


@functools.partial(
    pl.kernel,
    out_type=[
        jax.ShapeDtypeStruct((E,), _f32),
        jax.ShapeDtypeStruct((NC * N,), _f32),
    ],
    mesh=_MESH,
    compiler_params=pltpu.CompilerParams(needs_layout_passes=False),
    scratch_types=[
        pltpu.VMEM((N,), _f32),
        pltpu.VMEM((N,), _f32),
        pltpu.VMEM((EPW,), _i32),
        pltpu.VMEM((EPW,), _i32),
        pltpu.VMEM((EPW,), _i32),
        pltpu.VMEM((EPW,), _f32),
        pltpu.VMEM((16,), _f32),
        pltpu.VMEM((16,), _f32),
        pltpu.VMEM((CH,), _i32),
        pltpu.VMEM((SB,), _f32),
        pltpu.VMEM_SHARED((N,), _f32),
    ],
)
def _sc_att(hl_hbm, hr_hbm, head_hbm, tail_hbm, et_hbm, he_hbm, mub_hbm,
            ex_hbm, dpart_hbm,
            hl_v, hr_v, head_v, tail_v, et_v, ex_v,
            he_v, mub_v, hidx_v, zb_v, den_sh):
    c = lax.axis_index("c")
    s = lax.axis_index("s")
    wid = c * NS + s
    e0 = wid * EPW

    pltpu.sync_copy(hl_hbm, hl_v)
    pltpu.sync_copy(hr_hbm, hr_v)
    pltpu.sync_copy(head_hbm.at[pl.ds(e0, EPW)], head_v)
    pltpu.sync_copy(tail_hbm.at[pl.ds(e0, EPW)], tail_v)
    pltpu.sync_copy(et_hbm.at[pl.ds(e0, EPW)], et_v)
    pltpu.sync_copy(he_hbm, he_v)
    pltpu.sync_copy(mub_hbm, mub_v)
    mub = mub_v[pl.ds(0, 16)][0]

    def zb_step(i, _):
        zb_v[pl.ds(i * 16, 16)] = jnp.zeros((16,), _f32)
        return 0

    lax.fori_loop(0, SB // 16, zb_step, 0)

    @pl.when(s < NS - 1)
    def _():
        pltpu.sync_copy(zb_v, den_sh.at[pl.ds(s * SB, SB)])

    @pl.when(s == NS - 1)
    def _():
        pltpu.sync_copy(zb_v.at[pl.ds(0, SBL)], den_sh.at[pl.ds(s * SB, SBL)])

    def att_step(i, _):
        o = i * 16
        h = head_v[pl.ds(o, 16)]
        t = tail_v[pl.ds(o, 16)]
        g = et_v[pl.ds(o, 16)]
        ssum = (plsc.load_gather(hl_v, [h]) + plsc.load_gather(hr_v, [t])
                + plsc.load_gather(he_v, [g]))
        att = jnp.where(ssum > 0, ssum, 0.2 * ssum)
        ex_v[pl.ds(o, 16)] = jnp.exp(att - mub)
        return 0

    lax.fori_loop(0, EPW // 16, att_step, 0)
    pltpu.sync_copy(ex_v, ex_hbm.at[pl.ds(e0, EPW)])

    plsc.subcore_barrier()

    def den_step(i, _):
        o = i * CH
        for k in range(CH // 16):
            hidx_v[pl.ds(k * 16, 16)] = head_v[pl.ds(o + k * 16, 16)]
        pltpu.sync_copy(ex_v.at[pl.ds(o, CH)], den_sh.at[hidx_v], add=True)
        return 0

    lax.fori_loop(0, NCH, den_step, 0)

    plsc.subcore_barrier()

    @pl.when(s < NS - 1)
    def _():
        pltpu.sync_copy(den_sh.at[pl.ds(s * SB, SB)], zb_v)
        pltpu.sync_copy(zb_v, dpart_hbm.at[pl.ds(c * N + s * SB, SB)])

    @pl.when(s == NS - 1)
    def _():
        pltpu.sync_copy(den_sh.at[pl.ds(s * SB, SBL)], zb_v.at[pl.ds(0, SBL)])
        pltpu.sync_copy(zb_v.at[pl.ds(0, SBL)],
                        dpart_hbm.at[pl.ds(c * N + s * SB, SBL)])


SUP = 2000
NSUP = EPW // SUP
ZR = 64


@functools.partial(
    pl.kernel,
    out_type=jax.ShapeDtypeStruct((NC * N, D), _f32),
    mesh=_MESH,
    compiler_params=pltpu.CompilerParams(needs_layout_passes=False),
    scratch_types=[
        pltpu.VMEM((N,), _f32),
        pltpu.VMEM((2000,), _f32),
        pltpu.VMEM((SUP,), _i32),
        pltpu.VMEM((SUP,), _i32),
        pltpu.VMEM((SUP,), _f32),
        pltpu.VMEM((CH,), _i32),
        pltpu.VMEM((CH,), _i32),
        pltpu.VMEM((CH, D), _f32),
        pltpu.VMEM((ZR, D), _f32),
        pltpu.VMEM_SHARED((N, D), _f32),
        pltpu.SemaphoreType.DMA,
    ],
)
def _sc_agg(dpart_hbm, head_hbm, tail_hbm, ex_hbm, ht_hbm,
            opart_hbm,
            den_v, tmp_v, head_c, tail_c, ex_c, tidx_v, hidx_v, rows_v,
            zrow_v, out_sh, sem):
    c = lax.axis_index("c")
    s = lax.axis_index("s")
    wid = c * NS + s
    e0 = wid * EPW

    pltpu.sync_copy(dpart_hbm.at[pl.ds(0, N)], den_v)

    def dsum_outer(k, _):
        pltpu.sync_copy(dpart_hbm.at[pl.ds(N + k * 2000, 2000)], tmp_v)

        def dsum_step(i, _):
            o = i * 16
            ko = k * 2000 + o
            den_v[pl.ds(ko, 16)] = den_v[pl.ds(ko, 16)] + tmp_v[pl.ds(o, 16)]
            return 0

        lax.fori_loop(0, 2000 // 16, dsum_step, 0, unroll=2)
        return 0

    lax.fori_loop(0, N // 2000, dsum_outer, 0)

    def zr_step(i, _):
        r = i // 8
        j = i % 8
        zrow_v[r, pl.ds(j * 16, 16)] = jnp.zeros((16,), _f32)
        return 0

    lax.fori_loop(0, ZR * 8, zr_step, 0)

    @pl.when(s < NS - 1)
    def _():
        for k in range(SB // ZR):
            pltpu.sync_copy(zrow_v, out_sh.at[pl.ds(s * SB + k * ZR, ZR)])

    @pl.when(s == NS - 1)
    def _():
        for k in range(SBL // ZR):
            pltpu.sync_copy(zrow_v, out_sh.at[pl.ds(s * SB + k * ZR, ZR)])
        pltpu.sync_copy(zrow_v.at[pl.ds(0, SBL % ZR)],
                        out_sh.at[pl.ds(s * SB + (SBL // ZR) * ZR, SBL % ZR)])

    plsc.subcore_barrier()

    def super_step(sc_i, _):
        so = e0 + sc_i * SUP
        pltpu.sync_copy(head_hbm.at[pl.ds(so, SUP)], head_c)
        pltpu.sync_copy(tail_hbm.at[pl.ds(so, SUP)], tail_c)
        pltpu.sync_copy(ex_hbm.at[pl.ds(so, SUP)], ex_c)

        def alpha_step(i, _):
            o = i * 16
            h = head_c[pl.ds(o, 16)]
            den = plsc.load_gather(den_v, [h])
            ex_c[pl.ds(o, 16)] = ex_c[pl.ds(o, 16)] / jnp.maximum(den, 1e-16)
            return 0

        lax.fori_loop(0, SUP // 16, alpha_step, 0, unroll=2)

        def chunk_step(i, _):
            o = i * CH
            for k in range(CH // 16):
                tidx_v[pl.ds(k * 16, 16)] = tail_c[pl.ds(o + k * 16, 16)]
                hidx_v[pl.ds(k * 16, 16)] = head_c[pl.ds(o + k * 16, 16)]
            pltpu.async_copy(ht_hbm.at[tidx_v], rows_v, sem).wait()

            for g in range(CH // 16):
                av16 = ex_c[pl.ds(o + g * 16, 16)]
                for l in range(16):
                    av = jnp.full((16,), av16[l], _f32)
                    r = g * 16 + l
                    for j in range(D // 16):
                        rows_v[r, pl.ds(j * 16, 16)] = (
                            rows_v[r, pl.ds(j * 16, 16)] * av)

            pltpu.sync_copy(rows_v, out_sh.at[hidx_v], add=True)
            return 0

        lax.fori_loop(0, SUP // CH, chunk_step, 0)
        return 0

    lax.fori_loop(0, NSUP, super_step, 0)

    plsc.subcore_barrier()

    @pl.when(s < NS - 1)
    def _():
        for k in range(SB // ZR):
            r0 = s * SB + k * ZR
            pltpu.sync_copy(out_sh.at[pl.ds(r0, ZR)], zrow_v)
            pltpu.sync_copy(zrow_v, opart_hbm.at[pl.ds(c * N + r0, ZR)])

    @pl.when(s == NS - 1)
    def _():
        for k in range(SBL // ZR):
            r0 = s * SB + k * ZR
            pltpu.sync_copy(out_sh.at[pl.ds(r0, ZR)], zrow_v)
            pltpu.sync_copy(zrow_v, opart_hbm.at[pl.ds(c * N + r0, ZR)])
        r0 = s * SB + (SBL // ZR) * ZR
        pltpu.sync_copy(out_sh.at[pl.ds(r0, SBL % ZR)],
                        zrow_v.at[pl.ds(0, SBL % ZR)])
        pltpu.sync_copy(zrow_v.at[pl.ds(0, SBL % ZR)],
                        opart_hbm.at[pl.ds(c * N + r0, SBL % ZR)])


def _comb_body(p0_ref, p1_ref, o_ref):
    o_ref[...] = p0_ref[...] + p1_ref[...]


def _comb(p0, p1):
    blk = 1000
    return pl.pallas_call(
        _comb_body,
        grid=(N // blk,),
        in_specs=[
            pl.BlockSpec((blk, D), lambda i: (i, 0)),
            pl.BlockSpec((blk, D), lambda i: (i, 0)),
        ],
        out_specs=pl.BlockSpec((blk, D), lambda i: (i, 0)),
        out_shape=jax.ShapeDtypeStruct((N, D), _f32),
    )(p0, p1)


def kernel(head_feature, tail_feature, adj, tmp_edge, edge_emb, W, W_e,
           a_l, a_r, a_e):
    wt = W.T
    al = a_l.reshape(D, 1)
    ar = a_r.reshape(D, 1)
    ae16 = a_e.reshape(16)
    embt = jnp.zeros((16, 16), _f32).at[:3, :].set(edge_emb)
    head = adj[0].astype(_i32)
    tail = adj[1].astype(_i32)
    et = tmp_edge.astype(_i32)

    ht, hl, hr, he16, mub, _smax = _dense(head_feature, tail_feature, wt,
                                          al, ar, embt, W_e,
                                          ae16.reshape(1, 16))
    mub16 = jnp.broadcast_to(mub, (16,))
    ex, dpart = _sc_att(hl.reshape(N), hr.reshape(N), head, tail, et,
                        he16, mub16)
    opart = _sc_agg(dpart, head, tail, ex, ht)
    return _comb(opart[:N], opart[N:])


# speedup vs baseline: 33.6803x; 1.1891x over previous
import functools

import jax
import jax.numpy as jnp
from jax import lax
from jax.experimental import pallas as pl
from jax.experimental.pallas import tpu as pltpu
from jax.experimental.pallas import tpu_sc as plsc

N = 10000
E = 320000
D = 128
NC, NS = 2, 16
NW = NC * NS
EPW = E // NW
CH = 80
NCH = EPW // CH
SB = 640
SBL = N - (NS - 1) * SB

_f32 = jnp.float32
_i32 = jnp.int32

_MESH = plsc.VectorSubcoreMesh(core_axis_name="c", subcore_axis_name="s",
                               num_cores=NC, num_subcores=NS)


def _dense_body(hf_ref, tf_ref, wt_ref, al_ref, ar_ref, embt_ref, we_ref,
                ae_ref,
                ht_ref, hl_ref, hr_ref, he_ref, mub_ref, smax_ref):
    i = pl.program_id(0)
    ng = pl.num_programs(0)
    wt = wt_ref[...]
    ht = jnp.dot(tf_ref[...], wt, preferred_element_type=_f32)
    ht_ref[...] = ht
    hr = jnp.dot(ht, ar_ref[...], preferred_element_type=_f32)
    hr_ref[...] = hr
    wl = jnp.dot(wt, al_ref[...], preferred_element_type=_f32)
    hl = jnp.dot(hf_ref[...], wl, preferred_element_type=_f32)
    hl_ref[...] = hl
    cur = jnp.max(hl) + jnp.max(hr)

    @pl.when(i == 0)
    def _():
        smax_ref[0] = cur

    @pl.when(i > 0)
    def _():
        smax_ref[0] = jnp.maximum(smax_ref[0], cur)

    @pl.when(i == ng - 1)
    def _():
        e3 = jnp.dot(embt_ref[...], we_ref[...], preferred_element_type=_f32)
        he = jnp.sum(e3 * ae_ref[...], axis=1)
        he_ref[...] = he
        m = smax_ref[0] + jnp.max(he)
        mub_ref[0] = jnp.where(m > 0, m, 0.2 * m)


def _dense(hf, tf, wt, al, ar, embt, we, ae2):
    blk = 1000
    return pl.pallas_call(
        _dense_body,
        grid=(N // blk,),
        in_specs=[
            pl.BlockSpec((blk, D), lambda i: (i, 0)),
            pl.BlockSpec((blk, D), lambda i: (i, 0)),
            pl.BlockSpec((D, D), lambda i: (0, 0)),
            pl.BlockSpec((D, 1), lambda i: (0, 0)),
            pl.BlockSpec((D, 1), lambda i: (0, 0)),
            pl.BlockSpec((16, 16), lambda i: (0, 0)),
            pl.BlockSpec((16, 16), lambda i: (0, 0)),
            pl.BlockSpec((1, 16), lambda i: (0, 0)),
        ],
        out_specs=[
            pl.BlockSpec((blk, D), lambda i: (i, 0)),
            pl.BlockSpec((blk, 1), lambda i: (i, 0)),
            pl.BlockSpec((blk, 1), lambda i: (i, 0)),
            pl.BlockSpec((16,), lambda i: (0,)),
            pl.BlockSpec(memory_space=pltpu.SMEM),
            pl.BlockSpec(memory_space=pltpu.SMEM),
        ],
        out_shape=[
            jax.ShapeDtypeStruct((N, D), _f32),
            jax.ShapeDtypeStruct((N, 1), _f32),
            jax.ShapeDtypeStruct((N, 1), _f32),
            jax.ShapeDtypeStruct((16,), _f32),
            jax.ShapeDtypeStruct((1,), _f32),
            jax.ShapeDtypeStruct((1,), _f32),
        ],
    )(hf, tf, wt, al, ar, embt, we, ae2)


@functools.partial(
    pl.kernel,
    out_type=[
        jax.ShapeDtypeStruct((E,), _f32),
        jax.ShapeDtypeStruct((NC * N,), _f32),
    ],
    mesh=_MESH,
    compiler_params=pltpu.CompilerParams(needs_layout_passes=False),
    scratch_types=[
        pltpu.VMEM((N,), _f32),
        pltpu.VMEM((N,), _f32),
        pltpu.VMEM((EPW,), _i32),
        pltpu.VMEM((EPW,), _i32),
        pltpu.VMEM((EPW,), _i32),
        pltpu.VMEM((EPW,), _f32),
        pltpu.VMEM((16,), _f32),
        pltpu.VMEM((16,), _f32),
        pltpu.VMEM((CH,), _i32),
        pltpu.VMEM((SB,), _f32),
        pltpu.VMEM_SHARED((N,), _f32),
    ],
)
def _sc_att(hl_hbm, hr_hbm, head_hbm, tail_hbm, et_hbm, he_hbm, mub_hbm,
            ex_hbm, dpart_hbm,
            hl_v, hr_v, head_v, tail_v, et_v, ex_v,
            he_v, mub_v, hidx_v, zb_v, den_sh):
    c = lax.axis_index("c")
    s = lax.axis_index("s")
    wid = c * NS + s
    e0 = wid * EPW

    pltpu.sync_copy(hl_hbm, hl_v)
    pltpu.sync_copy(hr_hbm, hr_v)
    pltpu.sync_copy(head_hbm.at[pl.ds(e0, EPW)], head_v)
    pltpu.sync_copy(tail_hbm.at[pl.ds(e0, EPW)], tail_v)
    pltpu.sync_copy(et_hbm.at[pl.ds(e0, EPW)], et_v)
    pltpu.sync_copy(he_hbm, he_v)
    pltpu.sync_copy(mub_hbm, mub_v)
    mub = mub_v[pl.ds(0, 16)][0]

    def zb_step(i, _):
        zb_v[pl.ds(i * 16, 16)] = jnp.zeros((16,), _f32)
        return 0

    lax.fori_loop(0, SB // 16, zb_step, 0)

    @pl.when(s < NS - 1)
    def _():
        pltpu.sync_copy(zb_v, den_sh.at[pl.ds(s * SB, SB)])

    @pl.when(s == NS - 1)
    def _():
        pltpu.sync_copy(zb_v.at[pl.ds(0, SBL)], den_sh.at[pl.ds(s * SB, SBL)])

    def att_step(i, _):
        o = i * 16
        h = head_v[pl.ds(o, 16)]
        t = tail_v[pl.ds(o, 16)]
        g = et_v[pl.ds(o, 16)]
        ssum = (plsc.load_gather(hl_v, [h]) + plsc.load_gather(hr_v, [t])
                + plsc.load_gather(he_v, [g]))
        att = jnp.where(ssum > 0, ssum, 0.2 * ssum)
        ex_v[pl.ds(o, 16)] = jnp.exp(att - mub)
        return 0

    lax.fori_loop(0, EPW // 16, att_step, 0)
    pltpu.sync_copy(ex_v, ex_hbm.at[pl.ds(e0, EPW)])

    plsc.subcore_barrier()

    def den_step(i, _):
        o = i * CH
        for k in range(CH // 16):
            hidx_v[pl.ds(k * 16, 16)] = head_v[pl.ds(o + k * 16, 16)]
        pltpu.sync_copy(ex_v.at[pl.ds(o, CH)], den_sh.at[hidx_v], add=True)
        return 0

    lax.fori_loop(0, NCH, den_step, 0)

    plsc.subcore_barrier()

    @pl.when(s < NS - 1)
    def _():
        pltpu.sync_copy(den_sh.at[pl.ds(s * SB, SB)], zb_v)
        pltpu.sync_copy(zb_v, dpart_hbm.at[pl.ds(c * N + s * SB, SB)])

    @pl.when(s == NS - 1)
    def _():
        pltpu.sync_copy(den_sh.at[pl.ds(s * SB, SBL)], zb_v.at[pl.ds(0, SBL)])
        pltpu.sync_copy(zb_v.at[pl.ds(0, SBL)],
                        dpart_hbm.at[pl.ds(c * N + s * SB, SBL)])


SUP = 2000
NSUP = EPW // SUP
ZR = 64
CH2 = 40
NPAIR = SUP // (2 * CH2)


@functools.partial(
    pl.kernel,
    out_type=jax.ShapeDtypeStruct((NC * N, D), _f32),
    mesh=_MESH,
    compiler_params=pltpu.CompilerParams(needs_layout_passes=False),
    scratch_types=[
        pltpu.VMEM((N,), _f32),
        pltpu.VMEM((2000,), _f32),
        pltpu.VMEM((SUP,), _i32),
        pltpu.VMEM((SUP,), _i32),
        pltpu.VMEM((SUP,), _f32),
        pltpu.VMEM((CH2,), _i32),
        pltpu.VMEM((CH2,), _i32),
        pltpu.VMEM((CH2,), _i32),
        pltpu.VMEM((CH2,), _i32),
        pltpu.VMEM((CH2, D), _f32),
        pltpu.VMEM((CH2, D), _f32),
        pltpu.VMEM((ZR, D), _f32),
        pltpu.VMEM_SHARED((N, D), _f32),
        pltpu.SemaphoreType.DMA,
        pltpu.SemaphoreType.DMA,
        pltpu.SemaphoreType.DMA,
        pltpu.SemaphoreType.DMA,
    ],
)
def _sc_agg(dpart_hbm, head_hbm, tail_hbm, ex_hbm, ht_hbm,
            opart_hbm,
            den_v, tmp_v, head_c, tail_c, ex_c, tidx_a, hidx_a, tidx_b,
            hidx_b, rows_a, rows_b, zrow_v, out_sh, g_a, g_b, sc_a, sc_b):
    c = lax.axis_index("c")
    s = lax.axis_index("s")
    wid = c * NS + s
    e0 = wid * EPW

    pltpu.sync_copy(dpart_hbm.at[pl.ds(0, N)], den_v)

    def dsum_outer(k, _):
        pltpu.sync_copy(dpart_hbm.at[pl.ds(N + k * 2000, 2000)], tmp_v)

        def dsum_step(i, _):
            o = i * 16
            ko = k * 2000 + o
            den_v[pl.ds(ko, 16)] = den_v[pl.ds(ko, 16)] + tmp_v[pl.ds(o, 16)]
            return 0

        lax.fori_loop(0, 2000 // 16, dsum_step, 0, unroll=2)
        return 0

    lax.fori_loop(0, N // 2000, dsum_outer, 0)

    def zr_step(i, _):
        r = i // 8
        j = i % 8
        zrow_v[r, pl.ds(j * 16, 16)] = jnp.zeros((16,), _f32)
        return 0

    lax.fori_loop(0, ZR * 8, zr_step, 0)

    @pl.when(s < NS - 1)
    def _():
        for k in range(SB // ZR):
            pltpu.sync_copy(zrow_v, out_sh.at[pl.ds(s * SB + k * ZR, ZR)])

    @pl.when(s == NS - 1)
    def _():
        for k in range(SBL // ZR):
            pltpu.sync_copy(zrow_v, out_sh.at[pl.ds(s * SB + k * ZR, ZR)])
        pltpu.sync_copy(zrow_v.at[pl.ds(0, SBL % ZR)],
                        out_sh.at[pl.ds(s * SB + (SBL // ZR) * ZR, SBL % ZR)])

    plsc.subcore_barrier()

    def super_step(sc_i, _):
        so = e0 + sc_i * SUP
        pltpu.sync_copy(head_hbm.at[pl.ds(so, SUP)], head_c)
        pltpu.sync_copy(tail_hbm.at[pl.ds(so, SUP)], tail_c)
        pltpu.sync_copy(ex_hbm.at[pl.ds(so, SUP)], ex_c)

        def alpha_step(i, _):
            o = i * 16
            h = head_c[pl.ds(o, 16)]
            den = plsc.load_gather(den_v, [h])
            ex_c[pl.ds(o, 16)] = ex_c[pl.ds(o, 16)] / jnp.maximum(den, 1e-16)
            return 0

        lax.fori_loop(0, SUP // 16, alpha_step, 0, unroll=2)

        def stage(i, tidx_v, hidx_v):
            o = i * CH2
            for off in (0, 16, 24):
                tidx_v[pl.ds(off, 16)] = tail_c[pl.ds(o + off, 16)]
                hidx_v[pl.ds(off, 16)] = head_c[pl.ds(o + off, 16)]

        def scale(i, rows_v):
            o = i * CH2
            for base, lanes in ((0, range(16)), (16, range(16)),
                                (24, range(8, 16))):
                av16 = ex_c[pl.ds(o + base, 16)]
                for l in lanes:
                    r = base + l
                    av = jnp.full((16,), av16[l], _f32)
                    for jj in range(D // 16):
                        rows_v[r, pl.ds(jj * 16, 16)] = (
                            rows_v[r, pl.ds(jj * 16, 16)] * av)

        def g_start(tidx_v, rows_v, sem):
            pltpu.async_copy(ht_hbm.at[tidx_v], rows_v, sem)

        def g_wait(tidx_v, rows_v, sem):
            pltpu.make_async_copy(ht_hbm.at[tidx_v], rows_v, sem).wait()

        def s_start(rows_v, hidx_v, sem):
            pltpu.async_copy(rows_v, out_sh.at[hidx_v], sem, add=True)

        def s_wait(rows_v, hidx_v, sem):
            pltpu.make_async_copy(rows_v, out_sh.at[hidx_v], sem).wait()

        stage(0, tidx_a, hidx_a)
        g_start(tidx_a, rows_a, g_a)

        def pair(j, _):
            i0 = j * 2
            i1 = i0 + 1

            @pl.when(j > 0)
            def _():
                s_wait(rows_b, hidx_b, sc_b)

            stage(i1, tidx_b, hidx_b)
            g_start(tidx_b, rows_b, g_b)
            g_wait(tidx_a, rows_a, g_a)
            scale(i0, rows_a)
            s_start(rows_a, hidx_a, sc_a)

            g_wait(tidx_b, rows_b, g_b)
            scale(i1, rows_b)
            s_wait(rows_a, hidx_a, sc_a)

            @pl.when(j < NPAIR - 1)
            def _():
                stage(i0 + 2, tidx_a, hidx_a)
                g_start(tidx_a, rows_a, g_a)

            s_start(rows_b, hidx_b, sc_b)
            return 0

        lax.fori_loop(0, NPAIR, pair, 0)
        s_wait(rows_b, hidx_b, sc_b)
        return 0

    lax.fori_loop(0, NSUP, super_step, 0)

    plsc.subcore_barrier()

    @pl.when(s < NS - 1)
    def _():
        for k in range(SB // ZR):
            r0 = s * SB + k * ZR
            pltpu.sync_copy(out_sh.at[pl.ds(r0, ZR)], zrow_v)
            pltpu.sync_copy(zrow_v, opart_hbm.at[pl.ds(c * N + r0, ZR)])

    @pl.when(s == NS - 1)
    def _():
        for k in range(SBL // ZR):
            r0 = s * SB + k * ZR
            pltpu.sync_copy(out_sh.at[pl.ds(r0, ZR)], zrow_v)
            pltpu.sync_copy(zrow_v, opart_hbm.at[pl.ds(c * N + r0, ZR)])
        r0 = s * SB + (SBL // ZR) * ZR
        pltpu.sync_copy(out_sh.at[pl.ds(r0, SBL % ZR)],
                        zrow_v.at[pl.ds(0, SBL % ZR)])
        pltpu.sync_copy(zrow_v.at[pl.ds(0, SBL % ZR)],
                        opart_hbm.at[pl.ds(c * N + r0, SBL % ZR)])


def _comb_body(p0_ref, p1_ref, o_ref):
    o_ref[...] = p0_ref[...] + p1_ref[...]


def _comb(p0, p1):
    blk = 1000
    return pl.pallas_call(
        _comb_body,
        grid=(N // blk,),
        in_specs=[
            pl.BlockSpec((blk, D), lambda i: (i, 0)),
            pl.BlockSpec((blk, D), lambda i: (i, 0)),
        ],
        out_specs=pl.BlockSpec((blk, D), lambda i: (i, 0)),
        out_shape=jax.ShapeDtypeStruct((N, D), _f32),
    )(p0, p1)


def kernel(head_feature, tail_feature, adj, tmp_edge, edge_emb, W, W_e,
           a_l, a_r, a_e):
    wt = W.T
    al = a_l.reshape(D, 1)
    ar = a_r.reshape(D, 1)
    ae16 = a_e.reshape(16)
    embt = jnp.zeros((16, 16), _f32).at[:3, :].set(edge_emb)
    head = adj[0].astype(_i32)
    tail = adj[1].astype(_i32)
    et = tmp_edge.astype(_i32)

    ht, hl, hr, he16, mub, _smax = _dense(head_feature, tail_feature, wt,
                                          al, ar, embt, W_e,
                                          ae16.reshape(1, 16))
    mub16 = jnp.broadcast_to(mub, (16,))
    ex, dpart = _sc_att(hl.reshape(N), hr.reshape(N), head, tail, et,
                        he16, mub16)
    opart = _sc_agg(dpart, head, tail, ex, ht)
    return _comb(opart[:N], opart[N:])


# speedup vs baseline: 38.3810x; 1.1396x over previous
import functools

import jax
import jax.numpy as jnp
from jax import lax
from jax.experimental import pallas as pl
from jax.experimental.pallas import tpu as pltpu
from jax.experimental.pallas import tpu_sc as plsc

N = 10000
E = 320000
D = 128
NC, NS = 2, 16
NW = NC * NS
EPW = E // NW
CH = 80
NCH = EPW // CH
SB = 640
SBL = N - (NS - 1) * SB

_f32 = jnp.float32
_i32 = jnp.int32

_MESH = plsc.VectorSubcoreMesh(core_axis_name="c", subcore_axis_name="s",
                               num_cores=NC, num_subcores=NS)


def _dense_body(hf_ref, tf_ref, wt_ref, al_ref, ar_ref, embt_ref, we_ref,
                ae_ref,
                ht_ref, hl_ref, hr_ref, he_ref, mub_ref, smax_ref):
    i = pl.program_id(0)
    ng = pl.num_programs(0)
    wt = wt_ref[...]
    ht = jnp.dot(tf_ref[...], wt, preferred_element_type=_f32)
    ht_ref[...] = ht
    hr = jnp.dot(ht, ar_ref[...], preferred_element_type=_f32)
    hr_ref[...] = hr
    wl = jnp.dot(wt, al_ref[...], preferred_element_type=_f32)
    hl = jnp.dot(hf_ref[...], wl, preferred_element_type=_f32)
    hl_ref[...] = hl
    cur = jnp.max(hl) + jnp.max(hr)

    @pl.when(i == 0)
    def _():
        smax_ref[0] = cur

    @pl.when(i > 0)
    def _():
        smax_ref[0] = jnp.maximum(smax_ref[0], cur)

    @pl.when(i == ng - 1)
    def _():
        e3 = jnp.dot(embt_ref[...], we_ref[...], preferred_element_type=_f32)
        he = jnp.sum(e3 * ae_ref[...], axis=1)
        he_ref[...] = he
        m = smax_ref[0] + jnp.max(he)
        mub_ref[0] = jnp.where(m > 0, m, 0.2 * m)


def _dense(hf, tf, wt, al, ar, embt, we, ae2):
    blk = 1000
    return pl.pallas_call(
        _dense_body,
        grid=(N // blk,),
        in_specs=[
            pl.BlockSpec((blk, D), lambda i: (i, 0)),
            pl.BlockSpec((blk, D), lambda i: (i, 0)),
            pl.BlockSpec((D, D), lambda i: (0, 0)),
            pl.BlockSpec((D, 1), lambda i: (0, 0)),
            pl.BlockSpec((D, 1), lambda i: (0, 0)),
            pl.BlockSpec((16, 16), lambda i: (0, 0)),
            pl.BlockSpec((16, 16), lambda i: (0, 0)),
            pl.BlockSpec((1, 16), lambda i: (0, 0)),
        ],
        out_specs=[
            pl.BlockSpec((blk, D), lambda i: (i, 0)),
            pl.BlockSpec((blk, 1), lambda i: (i, 0)),
            pl.BlockSpec((blk, 1), lambda i: (i, 0)),
            pl.BlockSpec((16,), lambda i: (0,)),
            pl.BlockSpec(memory_space=pltpu.SMEM),
            pl.BlockSpec(memory_space=pltpu.SMEM),
        ],
        out_shape=[
            jax.ShapeDtypeStruct((N, D), _f32),
            jax.ShapeDtypeStruct((N, 1), _f32),
            jax.ShapeDtypeStruct((N, 1), _f32),
            jax.ShapeDtypeStruct((16,), _f32),
            jax.ShapeDtypeStruct((1,), _f32),
            jax.ShapeDtypeStruct((1,), _f32),
        ],
    )(hf, tf, wt, al, ar, embt, we, ae2)


@functools.partial(
    pl.kernel,
    out_type=[
        jax.ShapeDtypeStruct((E,), _f32),
        jax.ShapeDtypeStruct((NC * N,), _f32),
    ],
    mesh=_MESH,
    compiler_params=pltpu.CompilerParams(needs_layout_passes=False),
    scratch_types=[
        pltpu.VMEM((N,), _f32),
        pltpu.VMEM((N,), _f32),
        pltpu.VMEM((EPW,), _i32),
        pltpu.VMEM((EPW,), _i32),
        pltpu.VMEM((EPW,), _i32),
        pltpu.VMEM((EPW,), _f32),
        pltpu.VMEM((16,), _f32),
        pltpu.VMEM((16,), _f32),
        pltpu.VMEM((CH,), _i32),
        pltpu.VMEM((SB,), _f32),
        pltpu.VMEM_SHARED((N,), _f32),
    ],
)
def _sc_att(hl_hbm, hr_hbm, adj_hbm, et_hbm, he_hbm, mub_hbm,
            ex_hbm, dpart_hbm,
            hl_v, hr_v, head_v, tail_v, et_v, ex_v,
            he_v, mub_v, hidx_v, zb_v, den_sh):
    c = lax.axis_index("c")
    s = lax.axis_index("s")
    wid = c * NS + s
    e0 = wid * EPW

    pltpu.sync_copy(hl_hbm, hl_v)
    pltpu.sync_copy(hr_hbm, hr_v)
    pltpu.sync_copy(adj_hbm.at[pl.ds(e0, EPW)], head_v)
    pltpu.sync_copy(adj_hbm.at[pl.ds(E + e0, EPW)], tail_v)
    pltpu.sync_copy(et_hbm.at[pl.ds(e0, EPW)], et_v)
    pltpu.sync_copy(he_hbm, he_v)
    pltpu.sync_copy(mub_hbm, mub_v)
    mub = mub_v[pl.ds(0, 16)][0]

    def zb_step(i, _):
        zb_v[pl.ds(i * 16, 16)] = jnp.zeros((16,), _f32)
        return 0

    lax.fori_loop(0, SB // 16, zb_step, 0)

    @pl.when(s < NS - 1)
    def _():
        pltpu.sync_copy(zb_v, den_sh.at[pl.ds(s * SB, SB)])

    @pl.when(s == NS - 1)
    def _():
        pltpu.sync_copy(zb_v.at[pl.ds(0, SBL)], den_sh.at[pl.ds(s * SB, SBL)])

    def att_step(i, _):
        o = i * 16
        h = head_v[pl.ds(o, 16)]
        t = tail_v[pl.ds(o, 16)]
        g = et_v[pl.ds(o, 16)]
        ssum = (plsc.load_gather(hl_v, [h]) + plsc.load_gather(hr_v, [t])
                + plsc.load_gather(he_v, [g]))
        att = jnp.where(ssum > 0, ssum, 0.2 * ssum)
        ex_v[pl.ds(o, 16)] = jnp.exp(att - mub)
        return 0

    lax.fori_loop(0, EPW // 16, att_step, 0)
    pltpu.sync_copy(ex_v, ex_hbm.at[pl.ds(e0, EPW)])

    plsc.subcore_barrier()

    def den_step(i, _):
        o = i * CH
        for k in range(CH // 16):
            hidx_v[pl.ds(k * 16, 16)] = head_v[pl.ds(o + k * 16, 16)]
        pltpu.sync_copy(ex_v.at[pl.ds(o, CH)], den_sh.at[hidx_v], add=True)
        return 0

    lax.fori_loop(0, NCH, den_step, 0)

    plsc.subcore_barrier()

    @pl.when(s < NS - 1)
    def _():
        pltpu.sync_copy(den_sh.at[pl.ds(s * SB, SB)], zb_v)
        pltpu.sync_copy(zb_v, dpart_hbm.at[pl.ds(c * N + s * SB, SB)])

    @pl.when(s == NS - 1)
    def _():
        pltpu.sync_copy(den_sh.at[pl.ds(s * SB, SBL)], zb_v.at[pl.ds(0, SBL)])
        pltpu.sync_copy(zb_v.at[pl.ds(0, SBL)],
                        dpart_hbm.at[pl.ds(c * N + s * SB, SBL)])


SUP = 4000
NSUP = EPW // SUP
ZR = 32
CH2 = 80
NPAIR = SUP // (2 * CH2)


@functools.partial(
    pl.kernel,
    out_type=jax.ShapeDtypeStruct((NC * N, D), _f32),
    mesh=_MESH,
    compiler_params=pltpu.CompilerParams(needs_layout_passes=False),
    scratch_types=[
        pltpu.VMEM((N,), _f32),
        pltpu.VMEM((SUP,), _i32),
        pltpu.VMEM((SUP,), _i32),
        pltpu.VMEM((SUP,), _f32),
        pltpu.VMEM((CH2,), _i32),
        pltpu.VMEM((CH2,), _i32),
        pltpu.VMEM((CH2,), _i32),
        pltpu.VMEM((CH2,), _i32),
        pltpu.VMEM((CH2, D), _f32),
        pltpu.VMEM((CH2, D), _f32),
        pltpu.VMEM((ZR, D), _f32),
        pltpu.VMEM_SHARED((N, D), _f32),
        pltpu.SemaphoreType.DMA,
        pltpu.SemaphoreType.DMA,
        pltpu.SemaphoreType.DMA,
        pltpu.SemaphoreType.DMA,
    ],
)
def _sc_agg(dpart_hbm, adj_hbm, ex_hbm, ht_hbm,
            opart_hbm,
            den_v, head_c, tail_c, ex_c, tidx_a, hidx_a, tidx_b,
            hidx_b, rows_a, rows_b, zrow_v, out_sh, g_a, g_b, sc_a, sc_b):
    c = lax.axis_index("c")
    s = lax.axis_index("s")
    wid = c * NS + s
    e0 = wid * EPW

    pltpu.sync_copy(dpart_hbm.at[pl.ds(0, N)], den_v)

    def dsum_outer(k, _):
        pltpu.sync_copy(dpart_hbm.at[pl.ds(N + k * 2000, 2000)],
                        ex_c.at[pl.ds(0, 2000)])

        def dsum_step(i, _):
            o = i * 16
            ko = k * 2000 + o
            den_v[pl.ds(ko, 16)] = den_v[pl.ds(ko, 16)] + ex_c[pl.ds(o, 16)]
            return 0

        lax.fori_loop(0, 2000 // 16, dsum_step, 0, unroll=2)
        return 0

    lax.fori_loop(0, N // 2000, dsum_outer, 0)

    def zr_step(i, _):
        r = i // 8
        j = i % 8
        zrow_v[r, pl.ds(j * 16, 16)] = jnp.zeros((16,), _f32)
        return 0

    lax.fori_loop(0, ZR * 8, zr_step, 0)

    @pl.when(s < NS - 1)
    def _():
        for k in range(SB // ZR):
            pltpu.sync_copy(zrow_v, out_sh.at[pl.ds(s * SB + k * ZR, ZR)])

    @pl.when(s == NS - 1)
    def _():
        for k in range(SBL // ZR):
            pltpu.sync_copy(zrow_v, out_sh.at[pl.ds(s * SB + k * ZR, ZR)])
        pltpu.sync_copy(zrow_v.at[pl.ds(0, SBL % ZR)],
                        out_sh.at[pl.ds(s * SB + (SBL // ZR) * ZR, SBL % ZR)])

    plsc.subcore_barrier()

    def super_step(sc_i, _):
        so = e0 + sc_i * SUP
        pltpu.sync_copy(adj_hbm.at[pl.ds(so, SUP)], head_c)
        pltpu.sync_copy(adj_hbm.at[pl.ds(E + so, SUP)], tail_c)
        pltpu.sync_copy(ex_hbm.at[pl.ds(so, SUP)], ex_c)

        def alpha_step(i, _):
            o = i * 16
            h = head_c[pl.ds(o, 16)]
            den = plsc.load_gather(den_v, [h])
            ex_c[pl.ds(o, 16)] = ex_c[pl.ds(o, 16)] / jnp.maximum(den, 1e-16)
            return 0

        lax.fori_loop(0, SUP // 16, alpha_step, 0, unroll=2)

        def stage(i, tidx_v, hidx_v):
            o = i * CH2
            for off in range(0, CH2, 16):
                tidx_v[pl.ds(off, 16)] = tail_c[pl.ds(o + off, 16)]
                hidx_v[pl.ds(off, 16)] = head_c[pl.ds(o + off, 16)]

        def scale(i, rows_v):
            o = i * CH2
            for base in range(0, CH2, 16):
                av16 = ex_c[pl.ds(o + base, 16)]
                for l in range(16):
                    r = base + l
                    av = jnp.full((16,), av16[l], _f32)
                    for jj in range(D // 16):
                        rows_v[r, pl.ds(jj * 16, 16)] = (
                            rows_v[r, pl.ds(jj * 16, 16)] * av)

        def g_start(tidx_v, rows_v, sem):
            pltpu.async_copy(ht_hbm.at[tidx_v], rows_v, sem)

        def g_wait(tidx_v, rows_v, sem):
            pltpu.make_async_copy(ht_hbm.at[tidx_v], rows_v, sem).wait()

        def s_start(rows_v, hidx_v, sem):
            pltpu.async_copy(rows_v, out_sh.at[hidx_v], sem, add=True)

        def s_wait(rows_v, hidx_v, sem):
            pltpu.make_async_copy(rows_v, out_sh.at[hidx_v], sem).wait()

        stage(0, tidx_a, hidx_a)
        g_start(tidx_a, rows_a, g_a)

        def pair(j, _):
            i0 = j * 2
            i1 = i0 + 1

            @pl.when(j > 0)
            def _():
                s_wait(rows_b, hidx_b, sc_b)

            stage(i1, tidx_b, hidx_b)
            g_start(tidx_b, rows_b, g_b)
            g_wait(tidx_a, rows_a, g_a)
            scale(i0, rows_a)
            s_start(rows_a, hidx_a, sc_a)

            g_wait(tidx_b, rows_b, g_b)
            scale(i1, rows_b)
            s_wait(rows_a, hidx_a, sc_a)

            @pl.when(j < NPAIR - 1)
            def _():
                stage(i0 + 2, tidx_a, hidx_a)
                g_start(tidx_a, rows_a, g_a)

            s_start(rows_b, hidx_b, sc_b)
            return 0

        lax.fori_loop(0, NPAIR, pair, 0)
        s_wait(rows_b, hidx_b, sc_b)
        return 0

    lax.fori_loop(0, NSUP, super_step, 0)

    plsc.subcore_barrier()

    @pl.when(s < NS - 1)
    def _():
        for k in range(SB // ZR):
            r0 = s * SB + k * ZR
            pltpu.sync_copy(out_sh.at[pl.ds(r0, ZR)], zrow_v)
            pltpu.sync_copy(zrow_v, opart_hbm.at[pl.ds(c * N + r0, ZR)])

    @pl.when(s == NS - 1)
    def _():
        for k in range(SBL // ZR):
            r0 = s * SB + k * ZR
            pltpu.sync_copy(out_sh.at[pl.ds(r0, ZR)], zrow_v)
            pltpu.sync_copy(zrow_v, opart_hbm.at[pl.ds(c * N + r0, ZR)])
        r0 = s * SB + (SBL // ZR) * ZR
        pltpu.sync_copy(out_sh.at[pl.ds(r0, SBL % ZR)],
                        zrow_v.at[pl.ds(0, SBL % ZR)])
        pltpu.sync_copy(zrow_v.at[pl.ds(0, SBL % ZR)],
                        opart_hbm.at[pl.ds(c * N + r0, SBL % ZR)])


def _comb_body(p0_ref, p1_ref, o_ref):
    o_ref[...] = p0_ref[...] + p1_ref[...]


def _comb(opart):
    blk = 1000
    return pl.pallas_call(
        _comb_body,
        grid=(N // blk,),
        in_specs=[
            pl.BlockSpec((blk, D), lambda i: (i, 0)),
            pl.BlockSpec((blk, D), lambda i: (i + N // 1000, 0)),
        ],
        out_specs=pl.BlockSpec((blk, D), lambda i: (i, 0)),
        out_shape=jax.ShapeDtypeStruct((N, D), _f32),
    )(opart, opart)


def kernel(head_feature, tail_feature, adj, tmp_edge, edge_emb, W, W_e,
           a_l, a_r, a_e):
    wt = W.T
    al = a_l.reshape(D, 1)
    ar = a_r.reshape(D, 1)
    ae16 = a_e.reshape(16)
    embt = jnp.zeros((16, 16), _f32).at[:3, :].set(edge_emb)
    adjf = adj.reshape(2 * E).astype(_i32)
    et = tmp_edge.astype(_i32)

    ht, hl, hr, he16, mub, _smax = _dense(head_feature, tail_feature, wt,
                                          al, ar, embt, W_e,
                                          ae16.reshape(1, 16))
    mub16 = jnp.broadcast_to(mub, (16,))
    ex, dpart = _sc_att(hl.reshape(N), hr.reshape(N), adjf, et,
                        he16, mub16)
    opart = _sc_agg(dpart, adjf, ex, ht)
    return _comb(opart)
